# baseline (speedup 1.0000x reference)
import functools

import jax
import jax.numpy as jnp
from jax import lax
from jax.experimental import pallas as pl
from jax.experimental.pallas import tpu as pltpu
from jax.experimental.pallas import tpu_sc as plsc

_C = 256
_M = 128
_H = 512
_W = 512
_NW = 32
_CPT = _M // _NW
_RC = 64
_KC = _H // _RC
_NT = _CPT * _KC
_L = 16


def _sc_body(model_hbm, cm_hbm, out_hbm, cm_v, spbuf, gsem0, gsem1, gsem2,
             ssem0, ssem1, ssem2):
    cid = lax.axis_index("c")
    sid = lax.axis_index("s")
    wid = sid * 2 + cid
    ch0 = wid * _CPT

    iota = lax.broadcasted_iota(jnp.int32, (_L,), 0)
    cm_v[pl.ds(0, _L)] = jnp.minimum(ch0 + iota, _M - 1)
    pltpu.async_copy(cm_hbm.at[cm_v.at[pl.ds(0, _L)]],
                     cm_v.at[pl.ds(_L, _L)], gsem0).wait()
    cvec = cm_v[pl.ds(_L, _L)]
    cvals = [cvec[j] for j in range(_CPT)]

    def src_chan(n):
        j = lax.shift_right_logical(n, 3)
        c = cvals[0]
        for jj in range(1, _CPT):
            c = jnp.where(j == jj, cvals[jj], c)
        return c

    gsems = (gsem0, gsem1, gsem2)
    ssems = (ssem0, ssem1, ssem2)

    def g_start(n, slot):
        c = src_chan(n)
        r0 = jnp.bitwise_and(n, _KC - 1) * _RC
        pltpu.async_copy(model_hbm.at[c, pl.ds(r0, _RC)],
                         spbuf.at[sid, slot], gsems[slot])

    def g_wait(slot):
        pltpu.make_async_copy(model_hbm.at[0, pl.ds(0, _RC)],
                              spbuf.at[sid, slot], gsems[slot]).wait()

    def s_start(n, slot):
        o = ch0 + lax.shift_right_logical(n, 3)
        r0 = jnp.bitwise_and(n, _KC - 1) * _RC
        pltpu.async_copy(spbuf.at[sid, slot],
                         out_hbm.at[o, pl.ds(r0, _RC)], ssems[slot])

    def s_wait(slot):
        pltpu.make_async_copy(spbuf.at[sid, slot],
                              out_hbm.at[0, pl.ds(0, _RC)], ssems[slot]).wait()

    def step(n, slot):
        g_wait(slot)
        s_start(n, slot)

        @pl.when(n >= 1)
        def _():
            prev = (slot + 2) % 3
            @pl.when(n + 2 < _NT)
            def _():
                s_wait(prev)
                g_start(n + 2, prev)

    g_start(0, 0)
    g_start(1, 1)
    g_start(2, 2)

    def tri_body(i, carry):
        n0 = 3 * i
        step(n0, 0)
        step(n0 + 1, 1)
        step(n0 + 2, 2)
        return carry

    lax.fori_loop(0, _NT // 3, tri_body, 0)
    step(_NT - 2, (_NT - 2) % 3)
    step(_NT - 1, (_NT - 1) % 3)
    s_wait(0)
    s_wait(1)
    s_wait(2)


@jax.jit
def _sc_gather(model, channel_map):
    mesh = plsc.VectorSubcoreMesh(core_axis_name="c", subcore_axis_name="s")
    return pl.kernel(
        _sc_body,
        mesh=mesh,
        out_type=jax.ShapeDtypeStruct((_M, _H, _W), jnp.float32),
        scratch_types=[
            pltpu.VMEM((2 * _L,), jnp.int32),
            pltpu.VMEM_SHARED((16, 3, _RC, _W), jnp.float32),
            pltpu.SemaphoreType.DMA,
            pltpu.SemaphoreType.DMA,
            pltpu.SemaphoreType.DMA,
            pltpu.SemaphoreType.DMA,
            pltpu.SemaphoreType.DMA,
            pltpu.SemaphoreType.DMA,
        ],
    )(model, channel_map)


def kernel(model, channel_map):
    return _sc_gather(model, channel_map.astype(jnp.int32))


# speedup vs baseline: 1.0062x; 1.0062x over previous
import functools

import jax
import jax.numpy as jnp
from jax import lax
from jax.experimental import pallas as pl
from jax.experimental.pallas import tpu as pltpu
from jax.experimental.pallas import tpu_sc as plsc

_C = 256
_M = 128
_H = 512
_W = 512
_NW = 32
_CPT = _M // _NW
_RC = 64
_KC = _H // _RC
_NT = _CPT * _KC
_L = 16


def _sc_body(model_hbm, cm_hbm, out_hbm, cm_v, spbuf, gsem0, gsem1,
             ssem0, ssem1):
    cid = lax.axis_index("c")
    sid = lax.axis_index("s")
    wid = sid * 2 + cid
    ch0 = wid * _CPT

    iota = lax.broadcasted_iota(jnp.int32, (_L,), 0)
    cm_v[pl.ds(0, _L)] = jnp.minimum(ch0 + iota, _M - 1)
    pltpu.async_copy(cm_hbm.at[cm_v.at[pl.ds(0, _L)]],
                     cm_v.at[pl.ds(_L, _L)], gsem0).wait()
    cvec = cm_v[pl.ds(_L, _L)]
    cvals = [cvec[j] for j in range(_CPT)]

    def src_chan(n):
        j = lax.shift_right_logical(n, 3)
        c = cvals[0]
        for jj in range(1, _CPT):
            c = jnp.where(j == jj, cvals[jj], c)
        return c

    def g_start(n, slot, sem):
        c = src_chan(n)
        r0 = jnp.bitwise_and(n, _KC - 1) * _RC
        pltpu.async_copy(model_hbm.at[c, pl.ds(r0, _RC)],
                         spbuf.at[sid, slot], sem)

    def g_wait(slot, sem):
        pltpu.make_async_copy(model_hbm.at[0, pl.ds(0, _RC)],
                              spbuf.at[sid, slot], sem).wait()

    def s_start(n, slot, sem):
        o = ch0 + lax.shift_right_logical(n, 3)
        r0 = jnp.bitwise_and(n, _KC - 1) * _RC
        pltpu.async_copy(spbuf.at[sid, slot],
                         out_hbm.at[o, pl.ds(r0, _RC)], sem)

    def s_wait(slot, sem):
        pltpu.make_async_copy(spbuf.at[sid, slot],
                              out_hbm.at[0, pl.ds(0, _RC)], sem).wait()

    g_start(0, 0, gsem0)

    def pair_body(i, carry):
        n0 = 2 * i

        @pl.when(i > 0)
        def _():
            s_wait(1, ssem1)

        g_start(n0 + 1, 1, gsem1)
        g_wait(0, gsem0)
        s_start(n0, 0, ssem0)

        @pl.when(i < _NT // 2 - 1)
        def _():
            s_wait(0, ssem0)
            g_start(n0 + 2, 0, gsem0)

        g_wait(1, gsem1)
        s_start(n0 + 1, 1, ssem1)
        return carry

    lax.fori_loop(0, _NT // 2, pair_body, 0)
    s_wait(0, ssem0)
    s_wait(1, ssem1)


@jax.jit
def _sc_gather(model, channel_map):
    mesh = plsc.VectorSubcoreMesh(core_axis_name="c", subcore_axis_name="s")
    return pl.kernel(
        _sc_body,
        mesh=mesh,
        out_type=jax.ShapeDtypeStruct((_M, _H, _W), jnp.float32),
        scratch_types=[
            pltpu.VMEM((2 * _L,), jnp.int32),
            pltpu.VMEM_SHARED((16, 2, _RC, _W), jnp.float32),
            pltpu.SemaphoreType.DMA,
            pltpu.SemaphoreType.DMA,
            pltpu.SemaphoreType.DMA,
            pltpu.SemaphoreType.DMA,
        ],
    )(model, channel_map)


def kernel(model, channel_map):
    return _sc_gather(model, channel_map.astype(jnp.int32))


# speedup vs baseline: 1.0065x; 1.0002x over previous
import jax
import jax.numpy as jnp
from jax import lax
from jax.experimental import pallas as pl
from jax.experimental.pallas import tpu as pltpu
from jax.experimental.pallas import tpu_sc as plsc

_C = 256
_M = 128
_H = 512
_W = 512
_NW = 32
_CPT = _M // _NW
_RC = 64
_KC = _H // _RC
_NT = _CPT * _KC
_L = 16


def _sc_body(model_hbm, cm_hbm, out_hbm, cm_v, spbuf, gsem0, gsem1,
             ssem0, ssem1):
    cid = lax.axis_index("c")
    sid = lax.axis_index("s")
    wid = sid * 2 + cid
    ch0 = wid * _CPT

    iota = lax.broadcasted_iota(jnp.int32, (_L,), 0)
    cm_v[pl.ds(0, _L)] = jnp.minimum(ch0 + iota, _M - 1)
    pltpu.async_copy(cm_hbm.at[cm_v.at[pl.ds(0, _L)]],
                     cm_v.at[pl.ds(_L, _L)], gsem0).wait()
    cvec = cm_v[pl.ds(_L, _L)]
    cvals = [cvec[j] for j in range(_CPT)]

    def src_chan(n):
        j = lax.shift_right_logical(n, 3)
        c = cvals[0]
        for jj in range(1, _CPT):
            c = jnp.where(j == jj, cvals[jj], c)
        return c

    def g_start(n, slot, sem):
        c = src_chan(n)
        r0 = jnp.bitwise_and(n, _KC - 1) * _RC
        pltpu.async_copy(model_hbm.at[c, pl.ds(r0, _RC)],
                         spbuf.at[sid, slot], sem)

    def g_wait(slot, sem):
        pltpu.make_async_copy(model_hbm.at[0, pl.ds(0, _RC)],
                              spbuf.at[sid, slot], sem).wait()

    def s_start(n, slot, sem):
        o = ch0 + lax.shift_right_logical(n, 3)
        r0 = jnp.bitwise_and(n, _KC - 1) * _RC
        pltpu.async_copy(spbuf.at[sid, slot],
                         out_hbm.at[o, pl.ds(r0, _RC)], sem)

    def s_wait(slot, sem):
        pltpu.make_async_copy(spbuf.at[sid, slot],
                              out_hbm.at[0, pl.ds(0, _RC)], sem).wait()

    g_start(0, 0, gsem0)

    def pair_body(i, carry):
        n0 = 2 * i

        @pl.when(i > 0)
        def _():
            s_wait(1, ssem1)

        g_start(n0 + 1, 1, gsem1)
        g_wait(0, gsem0)
        s_start(n0, 0, ssem0)

        @pl.when(i < _NT // 2 - 1)
        def _():
            s_wait(0, ssem0)
            g_start(n0 + 2, 0, gsem0)

        g_wait(1, gsem1)
        s_start(n0 + 1, 1, ssem1)
        return carry

    lax.fori_loop(0, _NT // 2, pair_body, 0)
    s_wait(0, ssem0)
    s_wait(1, ssem1)


@jax.jit
def _sc_gather(model, channel_map):
    mesh = plsc.VectorSubcoreMesh(core_axis_name="c", subcore_axis_name="s")
    return pl.kernel(
        _sc_body,
        mesh=mesh,
        out_type=jax.ShapeDtypeStruct((_M, _H, _W), jnp.float32),
        scratch_types=[
            pltpu.VMEM((2 * _L,), jnp.int32),
            pltpu.VMEM_SHARED((16, 2, _RC, _W), jnp.float32),
            pltpu.SemaphoreType.DMA,
            pltpu.SemaphoreType.DMA,
            pltpu.SemaphoreType.DMA,
            pltpu.SemaphoreType.DMA,
        ],
    )(model, channel_map)


def kernel(model, channel_map):
    return _sc_gather(model, channel_map.astype(jnp.int32))
